# baseline (speedup 1.0000x reference)
import functools

import jax
import jax.numpy as jnp
from jax import lax
from jax.experimental import pallas as pl
from jax.experimental.pallas import tpu as pltpu
from jax.experimental.pallas import tpu_sc as plsc

N = 10000
D = 128
NC = 2
NS = 16
NT = NC * NS
CH = 128
EPT = 10240
NCH = EPT // CH
EP = NT * EPT
PADV = 10240
BSZ = 320
CAP = 512
NR = NT * BSZ
RB = 1000

_mesh = plsc.VectorSubcoreMesh(core_axis_name="c", subcore_axis_name="s")
_no_layout = pltpu.CompilerParams(needs_layout_passes=False)


@functools.partial(
    pl.kernel,
    out_type=jax.ShapeDtypeStruct((NR, 16), jnp.float32),
    mesh=_mesh,
    compiler_params=_no_layout,
    scratch_types=[
        pltpu.VMEM((BSZ + 8, 16), jnp.float32),
        pltpu.VMEM((NT * CAP,), jnp.int32),
    ],
)
def _count_sc(dstage_hbm, out_hbm, acc, didx):
    c = lax.axis_index("c")
    s = lax.axis_index("s")
    t = c * NS + s
    z16 = jnp.zeros((16,), jnp.float32)
    o16 = jnp.ones((16,), jnp.float32)

    def zero(i, _):
        acc[i, :] = z16
        return 0

    lax.fori_loop(0, BSZ + 8, zero, 0)
    pltpu.sync_copy(dstage_hbm.at[t], didx)

    def group(gi, _):
        l16 = didx[pl.ds(gi * 16, 16)]
        for e in range(16):
            r = jnp.squeeze(lax.slice(l16, (e,), (e + 1,)))
            plsc.addupdate(acc.at[r, :], o16)
        return 0

    lax.fori_loop(0, NT * CAP // 16, group, 0)
    pltpu.sync_copy(acc.at[pl.ds(0, BSZ)], out_hbm.at[pl.ds(t * BSZ, BSZ)])


@functools.partial(
    pl.kernel,
    out_type=[jax.ShapeDtypeStruct((NT, NT, CAP), jnp.int32),
              jax.ShapeDtypeStruct((NT, NT, CAP), jnp.int32)],
    mesh=_mesh,
    compiler_params=_no_layout,
    scratch_types=[
        pltpu.VMEM((EPT,), jnp.int32),
        pltpu.VMEM((EPT,), jnp.int32),
        pltpu.VMEM((EPT,), jnp.int32),
        pltpu.VMEM((NT * CAP,), jnp.int32),
        pltpu.VMEM((NT * CAP,), jnp.int32),
    ],
)
def _bucket_sc(src_hbm, dst_hbm, sout_hbm, dout_hbm, sidx, didx, bidx, sstg, dstg):
    c = lax.axis_index("c")
    s = lax.axis_index("s")
    w = c * NS + s
    pltpu.sync_copy(src_hbm.at[w], sidx)
    pltpu.sync_copy(dst_hbm.at[w], didx)

    z16 = jnp.zeros((16,), jnp.int32)
    p16 = jnp.full((16,), BSZ, jnp.int32)

    def prefill(i, _):
        sstg[pl.ds(i * 16, 16)] = z16
        dstg[pl.ds(i * 16, 16)] = p16
        return 0

    lax.fori_loop(0, NT * CAP // 16, prefill, 0)

    def prepass(i, _):
        d16 = didx[pl.ds(i * 16, 16)]
        b16 = d16 // BSZ
        bidx[pl.ds(i * 16, 16)] = b16
        didx[pl.ds(i * 16, 16)] = d16 - b16 * BSZ
        return 0

    lax.fori_loop(0, EPT // 16, prepass, 0)

    for b in range(NT):
        def bpass(i, cur, b=b):
            sl = pl.ds(i * 16, 16)
            m = bidx[sl] == b
            plsc.store_compressed(sstg.at[pl.ds(b * CAP + cur, 16)], sidx[sl], mask=m)
            plsc.store_compressed(dstg.at[pl.ds(b * CAP + cur, 16)], didx[sl], mask=m)
            pop = plsc.all_reduce_population_count(m)
            return cur + jnp.squeeze(lax.slice(pop, (0,), (1,)))

        lax.fori_loop(0, EPT // 16, bpass, jnp.int32(0))

    for b in range(NT):
        pltpu.sync_copy(sstg.at[pl.ds(b * CAP, CAP)], sout_hbm.at[b, w])
        pltpu.sync_copy(dstg.at[pl.ds(b * CAP, CAP)], dout_hbm.at[b, w])


_NCH2 = NT * CAP // CH


@functools.partial(
    pl.kernel,
    out_type=jax.ShapeDtypeStruct((NR, D), jnp.float32),
    mesh=_mesh,
    compiler_params=_no_layout,
    scratch_types=[
        pltpu.VMEM((BSZ + 8, D), jnp.float32),
        pltpu.VMEM((2, CH, D), jnp.float32),
        pltpu.VMEM((_NCH2, CH), jnp.int32),
        pltpu.VMEM((_NCH2 * CH,), jnp.int32),
        pltpu.SemaphoreType.DMA,
        pltpu.SemaphoreType.DMA,
    ],
)
def _segsum_sc(val_hbm, sstage_hbm, dstage_hbm, out_hbm, acc, gbuf, sidx, didx, sem0, sem1):
    c = lax.axis_index("c")
    s = lax.axis_index("s")
    t = c * NS + s
    z16 = jnp.zeros((16,), jnp.float32)

    def zero(i, _):
        for k in range(D // 16):
            acc[i, pl.ds(k * 16, 16)] = z16
        return 0

    lax.fori_loop(0, BSZ + 8, zero, 0)
    pltpu.sync_copy(sstage_hbm.at[t], sidx)
    pltpu.sync_copy(dstage_hbm.at[t], didx)

    def accum(p, j):
        def group(g, _):
            l16 = didx[pl.ds(j * CH + g * 16, 16)]
            for e in range(16):
                r = jnp.squeeze(lax.slice(l16, (e,), (e + 1,)))
                for k in range(D // 16):
                    v16 = gbuf[p, g * 16 + e, pl.ds(k * 16, 16)]
                    plsc.addupdate(acc.at[r, pl.ds(k * 16, 16)], v16)
            return 0

        lax.fori_loop(0, CH // 16, group, 0)

    pltpu.async_copy(val_hbm.at[sidx.at[0]], gbuf.at[0], sem0)

    def body(i, _):
        j0 = 2 * i
        pltpu.async_copy(val_hbm.at[sidx.at[j0 + 1]], gbuf.at[1], sem1)
        pltpu.make_async_copy(val_hbm.at[sidx.at[j0]], gbuf.at[0], sem0).wait()
        accum(0, j0)
        pltpu.async_copy(val_hbm.at[sidx.at[j0 + 2]], gbuf.at[0], sem0)
        pltpu.make_async_copy(val_hbm.at[sidx.at[j0 + 1]], gbuf.at[1], sem1).wait()
        accum(1, j0 + 1)
        return 0

    lax.fori_loop(0, (_NCH2 - 2) // 2, body, 0)
    pltpu.async_copy(val_hbm.at[sidx.at[_NCH2 - 1]], gbuf.at[1], sem1)
    pltpu.make_async_copy(val_hbm.at[sidx.at[_NCH2 - 2]], gbuf.at[0], sem0).wait()
    accum(0, _NCH2 - 2)
    pltpu.make_async_copy(val_hbm.at[sidx.at[_NCH2 - 1]], gbuf.at[1], sem1).wait()
    accum(1, _NCH2 - 1)

    pltpu.sync_copy(acc.at[pl.ds(0, BSZ)], out_hbm.at[pl.ds(t * BSZ, BSZ)])


def _row_specs(n_in):
    return [pl.BlockSpec((RB, D), lambda i: (i, 0)) for _ in range(n_in)]


_W_SPEC = pl.BlockSpec((D, D), lambda i: (0, 0))
_B_SPEC = pl.BlockSpec((1, D), lambda i: (0, 0))
_P_SPEC = pl.BlockSpec((RB, 16), lambda i: (i, 0))
_ROW = pl.BlockSpec((RB, D), lambda i: (i, 0))


def _mm_a(x, w0, p):
    def body(x_ref, w_ref, p_ref, y_ref, dinv_ref, sdiv_ref):
        cnt = p_ref[:, 0:1]
        dinv = lax.rsqrt(cnt + 1.0)
        sdiv = 1.0 / jnp.maximum(cnt, 1.0)
        xw = jnp.dot(x_ref[...], w_ref[...], preferred_element_type=jnp.float32)
        y_ref[...] = xw * dinv
        dinv_ref[...] = jnp.broadcast_to(dinv, (RB, D))
        sdiv_ref[...] = jnp.broadcast_to(sdiv, (RB, D))

    sh = jax.ShapeDtypeStruct((N, D), jnp.float32)
    return pl.pallas_call(
        body,
        grid=(N // RB,),
        in_specs=[_ROW, _W_SPEC, _P_SPEC],
        out_specs=[_ROW, _ROW, _ROW],
        out_shape=[sh, sh, sh],
    )(x, w0, p)


def _elem_b(g, y0, dinvb, b0):
    def body(g_ref, y_ref, dv_ref, b_ref, o_ref):
        t = dv_ref[...] * (g_ref[...] + y_ref[...]) + b_ref[...]
        o_ref[...] = jnp.maximum(t, 0.0)

    return pl.pallas_call(
        body,
        grid=(N // RB,),
        in_specs=_row_specs(3) + [_B_SPEC],
        out_specs=_ROW,
        out_shape=jax.ShapeDtypeStruct((N, D), jnp.float32),
    )(g, y0, dinvb, b0)


def _mm_c(g, sdivb, h, wl, wr, bl):
    def body(g_ref, sd_ref, h_ref, wl_ref, wr_ref, b_ref, o_ref):
        a = sd_ref[...] * g_ref[...]
        t = (jnp.dot(a, wl_ref[...], preferred_element_type=jnp.float32)
             + jnp.dot(h_ref[...], wr_ref[...], preferred_element_type=jnp.float32)
             + h_ref[...] + b_ref[...])
        o_ref[...] = jnp.maximum(t, 0.0)

    return pl.pallas_call(
        body,
        grid=(N // RB,),
        in_specs=_row_specs(3) + [_W_SPEC, _W_SPEC, _B_SPEC],
        out_specs=_ROW,
        out_shape=jax.ShapeDtypeStruct((N, D), jnp.float32),
    )(g, sdivb, h, wl, wr, bl)


def _mm_d(g, sdivb, h, wl, wr, bl, wlin, blin, wf, dinvb):
    def body(g_ref, sd_ref, h_ref, wl_ref, wr_ref, b_ref,
             wlin_ref, blin_ref, wf_ref, dv_ref, y_ref):
        a = sd_ref[...] * g_ref[...]
        t = (jnp.dot(a, wl_ref[...], preferred_element_type=jnp.float32)
             + jnp.dot(h_ref[...], wr_ref[...], preferred_element_type=jnp.float32)
             + h_ref[...] + b_ref[...])
        h3 = jnp.maximum(
            jnp.dot(t, wlin_ref[...], preferred_element_type=jnp.float32)
            + blin_ref[...], 0.0)
        y_ref[...] = jnp.dot(h3, wf_ref[...], preferred_element_type=jnp.float32) * dv_ref[...]

    return pl.pallas_call(
        body,
        grid=(N // RB,),
        in_specs=_row_specs(3) + [_W_SPEC, _W_SPEC, _B_SPEC, _W_SPEC, _B_SPEC,
                                  _W_SPEC] + _row_specs(1),
        out_specs=_ROW,
        out_shape=jax.ShapeDtypeStruct((N, D), jnp.float32),
    )(g, sdivb, h, wl, wr, bl, wlin, blin, wf, dinvb)


def _elem_e(g, y4, dinvb, bf):
    def body(g_ref, y_ref, dv_ref, b_ref, o_ref):
        o_ref[...] = dv_ref[...] * (g_ref[...] + y_ref[...]) + b_ref[...]

    return pl.pallas_call(
        body,
        grid=(N // RB,),
        in_specs=_row_specs(3) + [_B_SPEC],
        out_specs=_ROW,
        out_shape=jax.ShapeDtypeStruct((N, D), jnp.float32),
    )(g, y4, dinvb, bf)


def kernel(x, edge_index, W0, b0, Wl1, bl1, Wr1, Wl2, bl2, Wr2, Wlin, blin, Wf, bf):
    e = edge_index.shape[1]
    ppt = EPT - e // NT
    src = jnp.concatenate(
        [edge_index[0].reshape(NT, e // NT),
         jnp.zeros((NT, ppt), jnp.int32)], axis=1)
    dst = jnp.concatenate(
        [edge_index[1].reshape(NT, e // NT),
         jnp.full((NT, ppt), PADV, jnp.int32)], axis=1)
    ss, ds_ = _bucket_sc(src, dst)
    ss_g = ss.reshape(NT, _NCH2, CH)
    ds_g = ds_.reshape(NT, _NCH2 * CH)

    p = _count_sc(ds_g)
    y0, dinvb, sdivb = _mm_a(x, W0, p)

    g = _segsum_sc(y0, ss_g, ds_g)
    h1 = _elem_b(g[:N], y0, dinvb, b0.reshape(1, D))

    ga = _segsum_sc(h1, ss_g, ds_g)
    h2 = _mm_c(ga[:N], sdivb, h1, Wl1, Wr1, bl1.reshape(1, D))

    gb = _segsum_sc(h2, ss_g, ds_g)
    y4 = _mm_d(gb[:N], sdivb, h2, Wl2, Wr2, bl2.reshape(1, D),
               Wlin, blin.reshape(1, D), Wf, dinvb)

    gc = _segsum_sc(y4, ss_g, ds_g)
    return _elem_e(gc[:N], y4, dinvb, bf.reshape(1, D))


# speedup vs baseline: 14.3870x; 14.3870x over previous
import functools

import jax
import jax.numpy as jnp
from jax import lax
from jax.experimental import pallas as pl
from jax.experimental.pallas import tpu as pltpu
from jax.experimental.pallas import tpu_sc as plsc

N = 10000
D = 128
NC = 2
NS = 16
NT = NC * NS
CH = 128
IB = 16
NB = 5
NCH = IB * NB
EP = NT * NCH * CH
RPT = 632
NPAD = NS * RPT
RB = 1000

_mesh = plsc.VectorSubcoreMesh(core_axis_name="c", subcore_axis_name="s")


def _zero_rows(ref, nrows, ncols16):
    z = jnp.zeros((16,), jnp.float32)

    def body(i, _):
        for k in range(ncols16):
            ref[i, pl.ds(k * 16, 16)] = z
        return 0

    lax.fori_loop(0, nrows, body, 0)


def _fill_ones(ref, nrows):
    o = jnp.ones((16,), jnp.float32)

    def body(i, _):
        ref[i, :] = o
        return 0

    lax.fori_loop(0, nrows, body, 0)


def _zero_acc_slice(zsrc, acc, base):
    for k in range(4):
        pltpu.sync_copy(zsrc, acc.at[pl.ds(base + k * CH, CH)])
    pltpu.sync_copy(zsrc.at[pl.ds(0, RPT - 512)], acc.at[pl.ds(base + 512, RPT - 512)])


@functools.partial(
    pl.kernel,
    out_type=jax.ShapeDtypeStruct((NC, NPAD, 16), jnp.float32),
    mesh=_mesh,
    scratch_types=[
        pltpu.VMEM_SHARED((NPAD, 16), jnp.float32),
        pltpu.VMEM((CH, 16), jnp.float32),
        pltpu.VMEM((CH, 16), jnp.float32),
        pltpu.VMEM((NCH, CH), jnp.int32),
    ],
)
def _count_sc(dst_hbm, out_hbm, acc, ones_v, zeros_v, idx_v):
    c = lax.axis_index("c")
    s = lax.axis_index("s")
    w = c * NS + s
    _fill_ones(ones_v, CH)
    _zero_rows(zeros_v, CH, 1)
    base = s * RPT
    _zero_acc_slice(zeros_v, acc, base)
    plsc.subcore_barrier()
    pltpu.sync_copy(dst_hbm.at[w], idx_v)

    def body(j, _):
        pltpu.sync_copy(ones_v, acc.at[idx_v.at[j]], add=True)
        return 0

    lax.fori_loop(0, NCH, body, 0)
    plsc.subcore_barrier()
    pltpu.sync_copy(acc.at[pl.ds(base, RPT)], out_hbm.at[c, pl.ds(base, RPT)])


@functools.partial(
    pl.kernel,
    out_type=jax.ShapeDtypeStruct((NC, NPAD, D), jnp.float32),
    mesh=_mesh,
    scratch_types=[
        pltpu.VMEM_SHARED((NPAD, D), jnp.float32),
        pltpu.VMEM((2, CH, D), jnp.float32),
        pltpu.VMEM((2, IB, CH), jnp.int32),
        pltpu.VMEM((2, IB, CH), jnp.int32),
        pltpu.SemaphoreType.DMA,
        pltpu.SemaphoreType.DMA,
        pltpu.SemaphoreType.DMA,
        pltpu.SemaphoreType.DMA,
        pltpu.SemaphoreType.DMA,
        pltpu.SemaphoreType.DMA,
    ],
)
def _segsum_sc(val_hbm, src_hbm, dst_hbm, out_hbm,
               acc, gbuf, sidx, didx, sem0, sem1, semS0, semS1, semi0, semi1):
    c = lax.axis_index("c")
    s = lax.axis_index("s")
    w = c * NS + s
    _zero_rows(gbuf.at[0], CH, D // 16)
    base = s * RPT
    _zero_acc_slice(gbuf.at[0], acc, base)
    plsc.subcore_barrier()

    semi = (semi0, semi1)

    def load_idx(ib):
        p = ib % 2
        hs = pltpu.async_copy(src_hbm.at[w, pl.ds(ib * IB, IB)], sidx.at[p], semi[p])
        hd = pltpu.async_copy(dst_hbm.at[w, pl.ds(ib * IB, IB)], didx.at[p], semi[p])
        return hs, hd

    def process_block(sp, dp, sp_next):
        def body(i, _):
            j0 = 2 * i
            pltpu.make_async_copy(val_hbm.at[sp.at[j0]], gbuf.at[0], sem0).wait()
            pltpu.async_copy(gbuf.at[0], acc.at[dp.at[j0]], semS0, add=True)
            pltpu.make_async_copy(val_hbm.at[sp.at[j0 + 1]], gbuf.at[1], sem1).wait()
            pltpu.async_copy(gbuf.at[1], acc.at[dp.at[j0 + 1]], semS1, add=True)
            pltpu.make_async_copy(gbuf.at[0], acc.at[dp.at[j0]], semS0).wait()
            pltpu.async_copy(val_hbm.at[sp.at[j0 + 2]], gbuf.at[0], sem0)
            pltpu.make_async_copy(gbuf.at[1], acc.at[dp.at[j0 + 1]], semS1).wait()
            pltpu.async_copy(val_hbm.at[sp.at[j0 + 3]], gbuf.at[1], sem1)
            return 0

        lax.fori_loop(0, (IB - 2) // 2, body, 0)
        j0 = IB - 2
        pltpu.make_async_copy(val_hbm.at[sp.at[j0]], gbuf.at[0], sem0).wait()
        pltpu.async_copy(gbuf.at[0], acc.at[dp.at[j0]], semS0, add=True)
        pltpu.make_async_copy(val_hbm.at[sp.at[j0 + 1]], gbuf.at[1], sem1).wait()
        pltpu.async_copy(gbuf.at[1], acc.at[dp.at[j0 + 1]], semS1, add=True)
        pltpu.make_async_copy(gbuf.at[0], acc.at[dp.at[j0]], semS0).wait()
        pltpu.make_async_copy(gbuf.at[1], acc.at[dp.at[j0 + 1]], semS1).wait()
        if sp_next is not None:
            pltpu.async_copy(val_hbm.at[sp_next.at[0]], gbuf.at[0], sem0)
            pltpu.async_copy(val_hbm.at[sp_next.at[1]], gbuf.at[1], sem1)

    hs, hd = load_idx(0)
    hs.wait()
    hd.wait()
    pltpu.async_copy(val_hbm.at[sidx.at[0].at[0]], gbuf.at[0], sem0)
    pltpu.async_copy(val_hbm.at[sidx.at[0].at[1]], gbuf.at[1], sem1)
    for ib in range(NB):
        if ib + 1 < NB:
            nhs, nhd = load_idx(ib + 1)
            nhs.wait()
            nhd.wait()
            nxt = sidx.at[(ib + 1) % 2]
        else:
            nxt = None
        process_block(sidx.at[ib % 2], didx.at[ib % 2], nxt)
    plsc.subcore_barrier()
    pltpu.sync_copy(acc.at[pl.ds(base, RPT)], out_hbm.at[c, pl.ds(base, RPT)])


def _row_specs(n_in):
    return [pl.BlockSpec((RB, D), lambda i: (i, 0)) for _ in range(n_in)]


_W_SPEC = pl.BlockSpec((D, D), lambda i: (0, 0))
_B_SPEC = pl.BlockSpec((1, D), lambda i: (0, 0))
_P_SPEC = pl.BlockSpec((RB, 16), lambda i: (i, 0))
_ROW = pl.BlockSpec((RB, D), lambda i: (i, 0))


def _mm_a(x, w0, p0, p1):
    def body(x_ref, w_ref, p0_ref, p1_ref, y_ref, dinv_ref, sdiv_ref):
        cnt = p0_ref[:, 0:1] + p1_ref[:, 0:1]
        dinv = lax.rsqrt(cnt + 1.0)
        sdiv = 1.0 / jnp.maximum(cnt, 1.0)
        xw = jnp.dot(x_ref[...], w_ref[...], preferred_element_type=jnp.float32)
        y_ref[...] = xw * dinv
        dinv_ref[...] = jnp.broadcast_to(dinv, (RB, D))
        sdiv_ref[...] = jnp.broadcast_to(sdiv, (RB, D))

    sh = jax.ShapeDtypeStruct((N, D), jnp.float32)
    return pl.pallas_call(
        body,
        grid=(N // RB,),
        in_specs=[_ROW, _W_SPEC, _P_SPEC, _P_SPEC],
        out_specs=[_ROW, _ROW, _ROW],
        out_shape=[sh, sh, sh],
    )(x, w0, p0, p1)


def _elem_b(g0, g1, y0, dinvb, b0):
    def body(g0_ref, g1_ref, y_ref, dv_ref, b_ref, o_ref):
        t = dv_ref[...] * (g0_ref[...] + g1_ref[...] + y_ref[...]) + b_ref[...]
        o_ref[...] = jnp.maximum(t, 0.0)

    return pl.pallas_call(
        body,
        grid=(N // RB,),
        in_specs=_row_specs(4) + [_B_SPEC],
        out_specs=_ROW,
        out_shape=jax.ShapeDtypeStruct((N, D), jnp.float32),
    )(g0, g1, y0, dinvb, b0)


def _mm_c(g0, g1, sdivb, h, wl, wr, bl):
    def body(g0_ref, g1_ref, sd_ref, h_ref, wl_ref, wr_ref, b_ref, o_ref):
        a = sd_ref[...] * (g0_ref[...] + g1_ref[...])
        t = (jnp.dot(a, wl_ref[...], preferred_element_type=jnp.float32)
             + jnp.dot(h_ref[...], wr_ref[...], preferred_element_type=jnp.float32)
             + h_ref[...] + b_ref[...])
        o_ref[...] = jnp.maximum(t, 0.0)

    return pl.pallas_call(
        body,
        grid=(N // RB,),
        in_specs=_row_specs(4) + [_W_SPEC, _W_SPEC, _B_SPEC],
        out_specs=_ROW,
        out_shape=jax.ShapeDtypeStruct((N, D), jnp.float32),
    )(g0, g1, sdivb, h, wl, wr, bl)


def _mm_d(g0, g1, sdivb, h, wl, wr, bl, wlin, blin, wf, dinvb):
    def body(g0_ref, g1_ref, sd_ref, h_ref, wl_ref, wr_ref, b_ref,
             wlin_ref, blin_ref, wf_ref, dv_ref, y_ref):
        a = sd_ref[...] * (g0_ref[...] + g1_ref[...])
        t = (jnp.dot(a, wl_ref[...], preferred_element_type=jnp.float32)
             + jnp.dot(h_ref[...], wr_ref[...], preferred_element_type=jnp.float32)
             + h_ref[...] + b_ref[...])
        h3 = jnp.maximum(
            jnp.dot(t, wlin_ref[...], preferred_element_type=jnp.float32)
            + blin_ref[...], 0.0)
        y_ref[...] = jnp.dot(h3, wf_ref[...], preferred_element_type=jnp.float32) * dv_ref[...]

    return pl.pallas_call(
        body,
        grid=(N // RB,),
        in_specs=_row_specs(4) + [_W_SPEC, _W_SPEC, _B_SPEC, _W_SPEC, _B_SPEC,
                                  _W_SPEC] + _row_specs(1),
        out_specs=_ROW,
        out_shape=jax.ShapeDtypeStruct((N, D), jnp.float32),
    )(g0, g1, sdivb, h, wl, wr, bl, wlin, blin, wf, dinvb)


def _elem_e(g0, g1, y4, dinvb, bf):
    def body(g0_ref, g1_ref, y_ref, dv_ref, b_ref, o_ref):
        o_ref[...] = dv_ref[...] * (g0_ref[...] + g1_ref[...] + y_ref[...]) + b_ref[...]

    return pl.pallas_call(
        body,
        grid=(N // RB,),
        in_specs=_row_specs(4) + [_B_SPEC],
        out_specs=_ROW,
        out_shape=jax.ShapeDtypeStruct((N, D), jnp.float32),
    )(g0, g1, y4, dinvb, bf)


def kernel(x, edge_index, W0, b0, Wl1, bl1, Wr1, Wl2, bl2, Wr2, Wlin, blin, Wf, bf):
    e = edge_index.shape[1]
    pad = EP - e
    src = jnp.concatenate([edge_index[0], jnp.zeros((pad,), jnp.int32)])
    dst = jnp.concatenate([edge_index[1], jnp.full((pad,), N, jnp.int32)])
    src_r = src.reshape(NT, NCH, CH)
    dst_r = dst.reshape(NT, NCH, CH)

    p = _count_sc(dst_r)
    y0, dinvb, sdivb = _mm_a(x, W0, p[0], p[1])

    g = _segsum_sc(y0, src_r, dst_r)
    h1 = _elem_b(g[0], g[1], y0, dinvb, b0.reshape(1, D))

    ga = _segsum_sc(h1, src_r, dst_r)
    h2 = _mm_c(ga[0], ga[1], sdivb, h1, Wl1, Wr1, bl1.reshape(1, D))

    gb = _segsum_sc(h2, src_r, dst_r)
    y4 = _mm_d(gb[0], gb[1], sdivb, h2, Wl2, Wr2, bl2.reshape(1, D),
               Wlin, blin.reshape(1, D), Wf, dinvb)

    gc = _segsum_sc(y4, src_r, dst_r)
    return _elem_e(gc[0], gc[1], y4, dinvb, bf.reshape(1, D))


# speedup vs baseline: 15.9667x; 1.1098x over previous
import functools

import jax
import jax.numpy as jnp
from jax import lax
from jax.experimental import pallas as pl
from jax.experimental.pallas import tpu as pltpu
from jax.experimental.pallas import tpu_sc as plsc

N = 10000
D = 128
NC = 2
NS = 16
NT = NC * NS
CH = 128
IB = 16
NB = 5
NCH = IB * NB
EP = NT * NCH * CH
RPT = 632
NPAD = NS * RPT
RB = 1000

_mesh = plsc.VectorSubcoreMesh(core_axis_name="c", subcore_axis_name="s")


@functools.partial(
    pl.kernel,
    out_type=jax.ShapeDtypeStruct((NC, NPAD, 16), jnp.float32),
    mesh=_mesh,
    scratch_types=[
        pltpu.VMEM_SHARED((NPAD, 16), jnp.float32),
        pltpu.VMEM((CH, 16), jnp.float32),
        pltpu.VMEM((NCH, CH), jnp.int32),
    ],
)
def _count_sc(dst_hbm, ones_hbm, zeros_hbm, out_hbm, acc, ones_v, idx_v):
    c = lax.axis_index("c")
    s = lax.axis_index("s")
    w = c * NS + s
    pltpu.sync_copy(ones_hbm, ones_v)
    base = s * RPT
    pltpu.sync_copy(zeros_hbm, acc.at[pl.ds(base, RPT)])
    plsc.subcore_barrier()
    pltpu.sync_copy(dst_hbm.at[w], idx_v)

    def body(j, _):
        pltpu.sync_copy(ones_v, acc.at[idx_v.at[j]], add=True)
        return 0

    lax.fori_loop(0, NCH, body, 0)
    plsc.subcore_barrier()
    pltpu.sync_copy(acc.at[pl.ds(base, RPT)], out_hbm.at[c, pl.ds(base, RPT)])


@functools.partial(
    pl.kernel,
    out_type=jax.ShapeDtypeStruct((NC, NPAD, D), jnp.float32),
    mesh=_mesh,
    scratch_types=[
        pltpu.VMEM_SHARED((NPAD, D), jnp.float32),
        pltpu.VMEM((2, CH, D), jnp.float32),
        pltpu.VMEM((2, IB, CH), jnp.int32),
        pltpu.VMEM((2, IB, CH), jnp.int32),
        pltpu.SemaphoreType.DMA,
        pltpu.SemaphoreType.DMA,
        pltpu.SemaphoreType.DMA,
        pltpu.SemaphoreType.DMA,
    ],
)
def _segsum_sc(val_hbm, src_hbm, dst_hbm, zeros_hbm, out_hbm,
               acc, gbuf, sidx, didx, sem0, sem1, semi0, semi1):
    c = lax.axis_index("c")
    s = lax.axis_index("s")
    w = c * NS + s
    base = s * RPT
    pltpu.sync_copy(zeros_hbm, acc.at[pl.ds(base, RPT)])
    plsc.subcore_barrier()

    semi = (semi0, semi1)

    def load_idx(ib):
        p = ib % 2
        hs = pltpu.async_copy(src_hbm.at[w, pl.ds(ib * IB, IB)], sidx.at[p], semi[p])
        hd = pltpu.async_copy(dst_hbm.at[w, pl.ds(ib * IB, IB)], didx.at[p], semi[p])
        return hs, hd

    def process_block(sp, dp):
        pltpu.async_copy(val_hbm.at[sp.at[0]], gbuf.at[0], sem0)

        def body(i, _):
            j0 = 2 * i
            pltpu.async_copy(val_hbm.at[sp.at[j0 + 1]], gbuf.at[1], sem1)
            pltpu.make_async_copy(val_hbm.at[sp.at[j0]], gbuf.at[0], sem0).wait()
            pltpu.sync_copy(gbuf.at[0], acc.at[dp.at[j0]], add=True)
            pltpu.async_copy(val_hbm.at[sp.at[j0 + 2]], gbuf.at[0], sem0)
            pltpu.make_async_copy(val_hbm.at[sp.at[j0 + 1]], gbuf.at[1], sem1).wait()
            pltpu.sync_copy(gbuf.at[1], acc.at[dp.at[j0 + 1]], add=True)
            return 0

        lax.fori_loop(0, (IB - 2) // 2, body, 0)
        pltpu.async_copy(val_hbm.at[sp.at[IB - 1]], gbuf.at[1], sem1)
        pltpu.make_async_copy(val_hbm.at[sp.at[IB - 2]], gbuf.at[0], sem0).wait()
        pltpu.sync_copy(gbuf.at[0], acc.at[dp.at[IB - 2]], add=True)
        pltpu.make_async_copy(val_hbm.at[sp.at[IB - 1]], gbuf.at[1], sem1).wait()
        pltpu.sync_copy(gbuf.at[1], acc.at[dp.at[IB - 1]], add=True)

    hs, hd = load_idx(0)
    hs.wait()
    hd.wait()
    for ib in range(NB):
        if ib + 1 < NB:
            nhs, nhd = load_idx(ib + 1)
        process_block(sidx.at[ib % 2], didx.at[ib % 2])
        if ib + 1 < NB:
            nhs.wait()
            nhd.wait()
    plsc.subcore_barrier()
    pltpu.sync_copy(acc.at[pl.ds(base, RPT)], out_hbm.at[c, pl.ds(base, RPT)])


def _row_specs(n_in):
    return [pl.BlockSpec((RB, D), lambda i: (i, 0)) for _ in range(n_in)]


_W_SPEC = pl.BlockSpec((D, D), lambda i: (0, 0))
_B_SPEC = pl.BlockSpec((1, D), lambda i: (0, 0))
_P_SPEC = pl.BlockSpec((RB, 16), lambda i: (i, 0))
_ROW = pl.BlockSpec((RB, D), lambda i: (i, 0))


def _mm_a(x, w0, p0, p1):
    def body(x_ref, w_ref, p0_ref, p1_ref, y_ref, dinv_ref, sdiv_ref):
        cnt = p0_ref[:, 0:1] + p1_ref[:, 0:1]
        dinv = lax.rsqrt(cnt + 1.0)
        sdiv = 1.0 / jnp.maximum(cnt, 1.0)
        xw = jnp.dot(x_ref[...], w_ref[...], preferred_element_type=jnp.float32)
        y_ref[...] = xw * dinv
        dinv_ref[...] = jnp.broadcast_to(dinv, (RB, D))
        sdiv_ref[...] = jnp.broadcast_to(sdiv, (RB, D))

    sh = jax.ShapeDtypeStruct((N, D), jnp.float32)
    return pl.pallas_call(
        body,
        grid=(N // RB,),
        in_specs=[_ROW, _W_SPEC, _P_SPEC, _P_SPEC],
        out_specs=[_ROW, _ROW, _ROW],
        out_shape=[sh, sh, sh],
    )(x, w0, p0, p1)


def _elem_b(g0, g1, y0, dinvb, b0):
    def body(g0_ref, g1_ref, y_ref, dv_ref, b_ref, o_ref):
        t = dv_ref[...] * (g0_ref[...] + g1_ref[...] + y_ref[...]) + b_ref[...]
        o_ref[...] = jnp.maximum(t, 0.0)

    return pl.pallas_call(
        body,
        grid=(N // RB,),
        in_specs=_row_specs(4) + [_B_SPEC],
        out_specs=_ROW,
        out_shape=jax.ShapeDtypeStruct((N, D), jnp.float32),
    )(g0, g1, y0, dinvb, b0)


def _mm_c(g0, g1, sdivb, h, wl, wr, bl):
    def body(g0_ref, g1_ref, sd_ref, h_ref, wl_ref, wr_ref, b_ref, o_ref):
        a = sd_ref[...] * (g0_ref[...] + g1_ref[...])
        t = (jnp.dot(a, wl_ref[...], preferred_element_type=jnp.float32)
             + jnp.dot(h_ref[...], wr_ref[...], preferred_element_type=jnp.float32)
             + h_ref[...] + b_ref[...])
        o_ref[...] = jnp.maximum(t, 0.0)

    return pl.pallas_call(
        body,
        grid=(N // RB,),
        in_specs=_row_specs(4) + [_W_SPEC, _W_SPEC, _B_SPEC],
        out_specs=_ROW,
        out_shape=jax.ShapeDtypeStruct((N, D), jnp.float32),
    )(g0, g1, sdivb, h, wl, wr, bl)


def _mm_d(g0, g1, sdivb, h, wl, wr, bl, wlin, blin, wf, dinvb):
    def body(g0_ref, g1_ref, sd_ref, h_ref, wl_ref, wr_ref, b_ref,
             wlin_ref, blin_ref, wf_ref, dv_ref, y_ref):
        a = sd_ref[...] * (g0_ref[...] + g1_ref[...])
        t = (jnp.dot(a, wl_ref[...], preferred_element_type=jnp.float32)
             + jnp.dot(h_ref[...], wr_ref[...], preferred_element_type=jnp.float32)
             + h_ref[...] + b_ref[...])
        h3 = jnp.maximum(
            jnp.dot(t, wlin_ref[...], preferred_element_type=jnp.float32)
            + blin_ref[...], 0.0)
        y_ref[...] = jnp.dot(h3, wf_ref[...], preferred_element_type=jnp.float32) * dv_ref[...]

    return pl.pallas_call(
        body,
        grid=(N // RB,),
        in_specs=_row_specs(4) + [_W_SPEC, _W_SPEC, _B_SPEC, _W_SPEC, _B_SPEC,
                                  _W_SPEC] + _row_specs(1),
        out_specs=_ROW,
        out_shape=jax.ShapeDtypeStruct((N, D), jnp.float32),
    )(g0, g1, sdivb, h, wl, wr, bl, wlin, blin, wf, dinvb)


def _elem_e(g0, g1, y4, dinvb, bf):
    def body(g0_ref, g1_ref, y_ref, dv_ref, b_ref, o_ref):
        o_ref[...] = dv_ref[...] * (g0_ref[...] + g1_ref[...] + y_ref[...]) + b_ref[...]

    return pl.pallas_call(
        body,
        grid=(N // RB,),
        in_specs=_row_specs(4) + [_B_SPEC],
        out_specs=_ROW,
        out_shape=jax.ShapeDtypeStruct((N, D), jnp.float32),
    )(g0, g1, y4, dinvb, bf)


def kernel(x, edge_index, W0, b0, Wl1, bl1, Wr1, Wl2, bl2, Wr2, Wlin, blin, Wf, bf):
    e = edge_index.shape[1]
    pad = EP - e
    src = jnp.concatenate([edge_index[0], jnp.zeros((pad,), jnp.int32)])
    dst = jnp.concatenate([edge_index[1], jnp.full((pad,), N, jnp.int32)])
    src_r = src.reshape(NT, NCH, CH)
    dst_r = dst.reshape(NT, NCH, CH)
    zseg = jnp.zeros((RPT, D), jnp.float32)
    zcnt = jnp.zeros((RPT, 16), jnp.float32)
    ones = jnp.ones((CH, 16), jnp.float32)

    p = _count_sc(dst_r, ones, zcnt)
    y0, dinvb, sdivb = _mm_a(x, W0, p[0], p[1])

    g = _segsum_sc(y0, src_r, dst_r, zseg)
    h1 = _elem_b(g[0], g[1], y0, dinvb, b0.reshape(1, D))

    ga = _segsum_sc(h1, src_r, dst_r, zseg)
    h2 = _mm_c(ga[0], ga[1], sdivb, h1, Wl1, Wr1, bl1.reshape(1, D))

    gb = _segsum_sc(h2, src_r, dst_r, zseg)
    y4 = _mm_d(gb[0], gb[1], sdivb, h2, Wl2, Wr2, bl2.reshape(1, D),
               Wlin, blin.reshape(1, D), Wf, dinvb)

    gc = _segsum_sc(y4, src_r, dst_r, zseg)
    return _elem_e(gc[0], gc[1], y4, dinvb, bf.reshape(1, D))
